# baseline (speedup 1.0000x reference)
import functools

import jax
import jax.numpy as jnp
from jax import lax
from jax.experimental import pallas as pl
from jax.experimental.pallas import tpu as pltpu
from jax.experimental.pallas import tpu_sc as plsc

R = 2048
H = 2048
B = 16
NSUB = 16
ROWS = R // NSUB
CB = 32
NB = H // CB
STEPS = NB + NSUB - 1


def _body(ref_hbm, hyp_hbm, out_hbm, ref_v, hyp_v, row_v, bnd_in, bnd_out,
          corner_v, spmem):
    cid = lax.axis_index("c")
    sid = lax.axis_index("s")

    pltpu.sync_copy(ref_hbm.at[pl.ds(sid * (ROWS * B), ROWS * B)], ref_v)
    pltpu.sync_copy(hyp_hbm, hyp_v)

    base_m1 = (sid * ROWS).astype(jnp.float32)
    zero_vec = jnp.zeros((B,), jnp.float32)
    corner_v[...] = zero_vec + base_m1

    def init_row(r, _):
        row_v[pl.ds(r * B, B)] = zero_vec + (base_m1 + 1.0
                                             + r.astype(jnp.float32))
        return 0

    lax.fori_loop(0, ROWS, init_row, 0)

    def step(k, _):
        b = k - sid
        valid = jnp.logical_and(b >= 0, b < NB)

        @pl.when(jnp.logical_and(valid, sid > 0))
        def _():
            slot = ((k + 1) % 2) * NSUB + (sid - 1)
            pltpu.sync_copy(spmem.at[pl.ds(slot * (CB * B), CB * B)], bnd_in)

        @pl.when(jnp.logical_and(valid, sid == 0))
        def _():
            def fill(jj, _):
                bnd_in[pl.ds(jj * B, B)] = (
                    zero_vec + (b * CB + jj + 1).astype(jnp.float32))
                return 0
            lax.fori_loop(0, CB, fill, 0)

        @pl.when(valid)
        def _():
            def col_body(jj, corner):
                j = b * CB + jj
                hv = hyp_v[pl.ds(j * B, B)]
                topv = bnd_in[pl.ds(jj * B, B)]

                def cell(r, carry):
                    left, diag = carry
                    prev = row_v[pl.ds(r * B, B)]
                    c = jnp.where(ref_v[pl.ds(r * B, B)] == hv, 0.0, 1.0)
                    a = jnp.minimum(prev + 1.0, diag + c)
                    newv = jnp.minimum(left + 1.0, a)
                    row_v[pl.ds(r * B, B)] = newv
                    return newv, prev

                left, _unused = lax.fori_loop(0, ROWS, cell, (topv, corner),
                                              unroll=4)
                bnd_out[pl.ds(jj * B, B)] = left
                return topv

            corner = lax.fori_loop(0, CB, col_body, corner_v[...])
            corner_v[...] = corner
            slot = (k % 2) * NSUB + sid
            pltpu.sync_copy(bnd_out, spmem.at[pl.ds(slot * (CB * B), CB * B)])

        plsc.subcore_barrier()
        return 0

    lax.fori_loop(0, STEPS, step, 0)

    @pl.when(jnp.logical_and(cid == 0, sid == NSUB - 1))
    def _():
        pltpu.sync_copy(bnd_out.at[pl.ds((CB - 1) * B, B)], out_hbm)


@jax.jit
def kernel(ref, hyp):
    mesh = plsc.VectorSubcoreMesh(core_axis_name="c", subcore_axis_name="s")
    f = functools.partial(
        pl.kernel,
        mesh=mesh,
        out_type=jax.ShapeDtypeStruct((B,), jnp.float32),
        scratch_types=[
            pltpu.VMEM((ROWS * B,), jnp.int32),
            pltpu.VMEM((H * B,), jnp.int32),
            pltpu.VMEM((ROWS * B,), jnp.float32),
            pltpu.VMEM((CB * B,), jnp.float32),
            pltpu.VMEM((CB * B,), jnp.float32),
            pltpu.VMEM((B,), jnp.float32),
            pltpu.VMEM_SHARED((2 * NSUB * CB * B,), jnp.float32),
        ],
    )(_body)
    return f(ref.reshape(R * B), hyp.reshape(H * B))


# speedup vs baseline: 1.1971x; 1.1971x over previous
import functools

import jax
import jax.numpy as jnp
from jax import lax
from jax.experimental import pallas as pl
from jax.experimental.pallas import tpu as pltpu
from jax.experimental.pallas import tpu_sc as plsc

R = 2048
H = 2048
B = 16
NSUB = 16
ROWS = R // NSUB
CB = 32
NB = H // CB
STEPS = NB + NSUB - 1


def _body(ref_hbm, hyp_hbm, out_hbm, ref_v, hyp_v, row_v, bnd_in, bnd_out,
          corner_v, spmem):
    cid = lax.axis_index("c")
    sid = lax.axis_index("s")

    pltpu.sync_copy(ref_hbm.at[pl.ds(sid * (ROWS * B), ROWS * B)], ref_v)
    pltpu.sync_copy(hyp_hbm, hyp_v)

    base_m1 = (sid * ROWS).astype(jnp.float32)
    zero_vec = jnp.zeros((B,), jnp.float32)
    corner_v[...] = zero_vec + base_m1

    def init_row(r, _):
        row_v[pl.ds(r * B, B)] = zero_vec + (base_m1 + 1.0
                                             + r.astype(jnp.float32))
        return 0

    lax.fori_loop(0, ROWS, init_row, 0)

    def step(k, _):
        b = k - sid
        valid = jnp.logical_and(b >= 0, b < NB)

        @pl.when(jnp.logical_and(valid, sid > 0))
        def _():
            slot = ((k + 1) % 2) * NSUB + (sid - 1)
            pltpu.sync_copy(spmem.at[pl.ds(slot * (CB * B), CB * B)], bnd_in)

        @pl.when(jnp.logical_and(valid, sid == 0))
        def _():
            def fill(jj, _):
                bnd_in[pl.ds(jj * B, B)] = (
                    zero_vec + (b * CB + jj + 1).astype(jnp.float32))
                return 0
            lax.fori_loop(0, CB, fill, 0)

        @pl.when(valid)
        def _():
            def col_body(jj, corner):
                j = b * CB + jj
                hv = hyp_v[pl.ds(j * B, B)]
                topv = bnd_in[pl.ds(jj * B, B)]

                def cell(r, carry):
                    left, diag = carry
                    prev = row_v[pl.ds(r * B, B)]
                    c = jnp.where(ref_v[pl.ds(r * B, B)] == hv, 0.0, 1.0)
                    a = jnp.minimum(prev + 1.0, diag + c)
                    newv = jnp.minimum(left + 1.0, a)
                    row_v[pl.ds(r * B, B)] = newv
                    return newv, prev

                left, _unused = lax.fori_loop(0, ROWS, cell, (topv, corner))
                bnd_out[pl.ds(jj * B, B)] = left
                return topv

            corner = lax.fori_loop(0, CB, col_body, corner_v[...])
            corner_v[...] = corner
            slot = (k % 2) * NSUB + sid
            pltpu.sync_copy(bnd_out, spmem.at[pl.ds(slot * (CB * B), CB * B)])

        plsc.subcore_barrier()
        return 0

    lax.fori_loop(0, STEPS, step, 0)

    @pl.when(jnp.logical_and(cid == 0, sid == NSUB - 1))
    def _():
        pltpu.sync_copy(bnd_out.at[pl.ds((CB - 1) * B, B)], out_hbm)


@jax.jit
def kernel(ref, hyp):
    mesh = plsc.VectorSubcoreMesh(core_axis_name="c", subcore_axis_name="s")
    f = functools.partial(
        pl.kernel,
        mesh=mesh,
        out_type=jax.ShapeDtypeStruct((B,), jnp.float32),
        scratch_types=[
            pltpu.VMEM((ROWS * B,), jnp.int32),
            pltpu.VMEM((H * B,), jnp.int32),
            pltpu.VMEM((ROWS * B,), jnp.float32),
            pltpu.VMEM((CB * B,), jnp.float32),
            pltpu.VMEM((CB * B,), jnp.float32),
            pltpu.VMEM((B,), jnp.float32),
            pltpu.VMEM_SHARED((2 * NSUB * CB * B,), jnp.float32),
        ],
    )(_body)
    return f(ref.reshape(R * B), hyp.reshape(H * B))


# speedup vs baseline: 2.4726x; 2.0654x over previous
import functools

import jax
import jax.numpy as jnp
from jax import lax
from jax.experimental import pallas as pl
from jax.experimental.pallas import tpu as pltpu
from jax.experimental.pallas import tpu_sc as plsc

R = 2048
H = 2048
B = 16
NSUB = 16
ROWS = R // NSUB
CB = 32
K = 4
NB = H // CB
STEPS = NB + NSUB - 1


def _body(ref_hbm, hyp_hbm, out_hbm, ref_v, hyp_v, row_v, bnd_in, bnd_out,
          corner_v, spmem):
    cid = lax.axis_index("c")
    sid = lax.axis_index("s")

    pltpu.sync_copy(ref_hbm.at[pl.ds(sid * (ROWS * B), ROWS * B)], ref_v)
    pltpu.sync_copy(hyp_hbm, hyp_v)

    base_m1 = (sid * ROWS).astype(jnp.float32)
    zero_vec = jnp.zeros((B,), jnp.float32)
    corner_v[...] = zero_vec + base_m1

    def init_row(r, _):
        row_v[pl.ds(r * B, B)] = zero_vec + (base_m1 + 1.0
                                             + r.astype(jnp.float32))
        return 0

    lax.fori_loop(0, ROWS, init_row, 0)

    def step(k, _):
        b = k - sid
        valid = jnp.logical_and(b >= 0, b < NB)

        @pl.when(jnp.logical_and(valid, sid > 0))
        def _():
            slot = ((k + 1) % 2) * NSUB + (sid - 1)
            pltpu.sync_copy(spmem.at[pl.ds(slot * (CB * B), CB * B)], bnd_in)

        @pl.when(jnp.logical_and(valid, sid == 0))
        def _():
            def fill(jj, _):
                bnd_in[pl.ds(jj * B, B)] = (
                    zero_vec + (b * CB + jj + 1).astype(jnp.float32))
                return 0
            lax.fori_loop(0, CB, fill, 0)

        @pl.when(valid)
        def _():
            def group_body(g, corner):
                jj0 = g * K
                jcol0 = b * CB + jj0
                hv = [hyp_v[pl.ds((jcol0 + q) * B, B)] for q in range(K)]
                topv = [bnd_in[pl.ds((jj0 + q) * B, B)] for q in range(K)]
                diag0 = [corner] + topv[:K - 1]

                def cell(q, r, left, diag, prev):
                    c = jnp.where(ref_v[pl.ds(r * B, B)] == hv[q], 0.0, 1.0)
                    a = jnp.minimum(prev + 1.0, diag + c)
                    newv = jnp.minimum(left + 1.0, a)
                    if q == K - 1:
                        row_v[pl.ds(r * B, B)] = newv
                    return newv

                lefts = [None] * K
                diags = [None] * K
                fwds = [None] * (K - 1)
                for t in range(K):
                    fwds_old = list(fwds)
                    for q in range(t + 1):
                        r = t - q
                        left = topv[q] if r == 0 else lefts[q]
                        diag = diag0[q] if r == 0 else diags[q]
                        prev = (row_v[pl.ds(r * B, B)] if q == 0
                                else fwds_old[q - 1])
                        newv = cell(q, r, left, diag, prev)
                        lefts[q] = newv
                        diags[q] = prev
                        if q < K - 1:
                            fwds[q] = newv

                def tstep(t, carry):
                    lefts, diags, fwds = carry
                    nl, nd, nf = [], [], []
                    for q in range(K):
                        r = t - q
                        prev = (row_v[pl.ds(r * B, B)] if q == 0
                                else fwds[q - 1])
                        newv = cell(q, r, lefts[q], diags[q], prev)
                        nl.append(newv)
                        nd.append(prev)
                        if q < K - 1:
                            nf.append(newv)
                    return tuple(nl), tuple(nd), tuple(nf)

                lefts, diags, fwds = lax.fori_loop(
                    K, ROWS - 1, tstep,
                    (tuple(lefts), tuple(diags), tuple(fwds)))
                lefts, diags, fwds = list(lefts), list(diags), list(fwds)

                for t in range(ROWS - 1, ROWS - 1 + K):
                    fwds_old = list(fwds)
                    for q in range(max(0, t - (ROWS - 1)), K):
                        r = t - q
                        prev = (row_v[pl.ds(r * B, B)] if q == 0
                                else fwds_old[q - 1])
                        newv = cell(q, r, lefts[q], diags[q], prev)
                        lefts[q] = newv
                        diags[q] = prev
                        if q < K - 1:
                            fwds[q] = newv
                        if r == ROWS - 1:
                            bnd_out[pl.ds((jj0 + q) * B, B)] = newv
                return topv[K - 1]

            corner = lax.fori_loop(0, CB // K, group_body, corner_v[...])
            corner_v[...] = corner
            slot = (k % 2) * NSUB + sid
            pltpu.sync_copy(bnd_out, spmem.at[pl.ds(slot * (CB * B), CB * B)])

        plsc.subcore_barrier()
        return 0

    lax.fori_loop(0, STEPS, step, 0)

    @pl.when(jnp.logical_and(cid == 0, sid == NSUB - 1))
    def _():
        pltpu.sync_copy(bnd_out.at[pl.ds((CB - 1) * B, B)], out_hbm)


@jax.jit
def kernel(ref, hyp):
    mesh = plsc.VectorSubcoreMesh(core_axis_name="c", subcore_axis_name="s")
    f = functools.partial(
        pl.kernel,
        mesh=mesh,
        out_type=jax.ShapeDtypeStruct((B,), jnp.float32),
        scratch_types=[
            pltpu.VMEM((ROWS * B,), jnp.int32),
            pltpu.VMEM((H * B,), jnp.int32),
            pltpu.VMEM((ROWS * B,), jnp.float32),
            pltpu.VMEM((CB * B,), jnp.float32),
            pltpu.VMEM((CB * B,), jnp.float32),
            pltpu.VMEM((B,), jnp.float32),
            pltpu.VMEM_SHARED((2 * NSUB * CB * B,), jnp.float32),
        ],
    )(_body)
    return f(ref.reshape(R * B), hyp.reshape(H * B))


# speedup vs baseline: 22.5585x; 9.1236x over previous
import functools

import jax
import jax.numpy as jnp
from jax import lax
from jax.experimental import pallas as pl
from jax.experimental.pallas import tpu as pltpu
from jax.experimental.pallas import tpu_sc as plsc

R = 2048
H = 2048
B = 16
VOCAB = 1000
NSUB = 16
ROWS = R // NSUB
W = ROWS // 32
CB = 32
NB = H // CB
STEPS = NB + NSUB - 1


def _column_step(vp, vn, eq, hinp, hinn):
    x = [eq[w] | vn[w] for w in range(W)]
    x[0] = x[0] | hinn
    d0 = [None] * W
    carry = None
    for w in range(W):
        a = vp[w]
        bb = x[w] & a
        t = a + bb
        s = t if carry is None else t + carry
        if w < W - 1:
            carry = lax.shift_right_logical(
                (a & bb) | ((a | bb) & ~s), 31)
        d0[w] = (s ^ a) | x[w]
    hn = [vp[w] & d0[w] for w in range(W)]
    hp = [vn[w] | ~(vp[w] | d0[w]) for w in range(W)]
    houtp = lax.shift_right_logical(hp[W - 1], 31)
    houtn = lax.shift_right_logical(hn[W - 1], 31)
    vp2 = [None] * W
    vn2 = [None] * W
    upp, upn = hinp, hinn
    for w in range(W):
        shp = (hp[w] << 1) | upp
        shn = (hn[w] << 1) | upn
        if w < W - 1:
            upp = lax.shift_right_logical(hp[w], 31)
            upn = lax.shift_right_logical(hn[w], 31)
        vp2[w] = shn | ~(shp | d0[w])
        vn2[w] = shp & d0[w]
    return vp2, vn2, houtp, houtn


def _body(ref_hbm, hyp_hbm, out_hbm, ref_v, hyp_v, peq_v, vpn_v, score_v,
          bnd_in, bnd_out, out_v, spmem):
    cid = lax.axis_index("c")
    sid = lax.axis_index("s")
    iota = lax.iota(jnp.int32, 16)
    zero = jnp.zeros((B,), jnp.int32)
    ones = zero - 1

    pltpu.sync_copy(ref_hbm.at[pl.ds(sid * (ROWS * B), ROWS * B)], ref_v)
    pltpu.sync_copy(hyp_hbm, hyp_v)

    def zero_peq(i, _):
        peq_v[pl.ds(i * B, B)] = zero
        return 0

    lax.fori_loop(0, VOCAB * W, zero_peq, 0, unroll=8)

    for w in range(W):
        def set_bit(r2, _, w=w):
            sym = ref_v[pl.ds((w * 32 + r2) * B, B)]
            idx = (sym * (W * B)) + (w * B) + iota
            bit = (zero + 1) << r2
            cur = plsc.load_gather(peq_v, [idx])
            plsc.store_scatter(peq_v, [idx], cur | bit)
            return 0

        lax.fori_loop(0, 32, set_bit, 0)

    for w in range(W):
        vpn_v[pl.ds(w * B, B)] = ones
        vpn_v[pl.ds((W + w) * B, B)] = zero
    score_v[...] = zero + R

    def step(k, _):
        b = k - sid
        valid = jnp.logical_and(b >= 0, b < NB)

        @pl.when(jnp.logical_and(valid, sid > 0))
        def _():
            slot = ((k + 1) % 2) * NSUB + (sid - 1)
            pltpu.sync_copy(spmem.at[pl.ds(slot * (2 * B), 2 * B)], bnd_in)

        @pl.when(jnp.logical_and(valid, sid == 0))
        def _():
            bnd_in[pl.ds(0, B)] = ones
            bnd_in[pl.ds(B, B)] = zero

        @pl.when(valid)
        def _():
            hinp_pack = bnd_in[pl.ds(0, B)]
            hinn_pack = bnd_in[pl.ds(B, B)]
            vp = [vpn_v[pl.ds(w * B, B)] for w in range(W)]
            vn = [vpn_v[pl.ds((W + w) * B, B)] for w in range(W)]

            def col(jj, carry):
                vp, vn, score, outp, outn = carry
                hv = hyp_v[pl.ds((b * CB + jj) * B, B)]
                hinp = lax.shift_right_logical(hinp_pack, jj) & 1
                hinn = lax.shift_right_logical(hinn_pack, jj) & 1
                idx0 = (hv * (W * B)) + iota
                eq = [plsc.load_gather(peq_v, [idx0 + (w * B)])
                      for w in range(W)]
                vp, vn, hop, hon = _column_step(vp, vn, eq, hinp, hinn)
                score = score + hop - hon
                outp = outp | (hop << jj)
                outn = outn | (hon << jj)
                return tuple(vp), tuple(vn), score, outp, outn

            vp, vn, score, outp, outn = lax.fori_loop(
                0, CB, col,
                (tuple(vp), tuple(vn), score_v[...], zero, zero))

            for w in range(W):
                vpn_v[pl.ds(w * B, B)] = vp[w]
                vpn_v[pl.ds((W + w) * B, B)] = vn[w]
            score_v[...] = score
            bnd_out[pl.ds(0, B)] = outp
            bnd_out[pl.ds(B, B)] = outn
            slot = (k % 2) * NSUB + sid
            pltpu.sync_copy(bnd_out, spmem.at[pl.ds(slot * (2 * B), 2 * B)])

        plsc.subcore_barrier()
        return 0

    lax.fori_loop(0, STEPS, step, 0)

    @pl.when(jnp.logical_and(cid == 0, sid == NSUB - 1))
    def _():
        out_v[...] = score_v[...].astype(jnp.float32)
        pltpu.sync_copy(out_v, out_hbm)


@jax.jit
def kernel(ref, hyp):
    mesh = plsc.VectorSubcoreMesh(core_axis_name="c", subcore_axis_name="s")
    f = functools.partial(
        pl.kernel,
        mesh=mesh,
        compiler_params=pltpu.CompilerParams(needs_layout_passes=False),
        out_type=jax.ShapeDtypeStruct((B,), jnp.float32),
        scratch_types=[
            pltpu.VMEM((ROWS * B,), jnp.int32),
            pltpu.VMEM((H * B,), jnp.int32),
            pltpu.VMEM((VOCAB * W * B,), jnp.int32),
            pltpu.VMEM((2 * W * B,), jnp.int32),
            pltpu.VMEM((B,), jnp.int32),
            pltpu.VMEM((2 * B,), jnp.int32),
            pltpu.VMEM((2 * B,), jnp.int32),
            pltpu.VMEM((B,), jnp.float32),
            pltpu.VMEM_SHARED((2 * NSUB * 2 * B,), jnp.int32),
        ],
    )(_body)
    return f(ref.reshape(R * B), hyp.reshape(H * B))
